# baseline (speedup 1.0000x reference)
import jax
import jax.numpy as jnp
from jax import lax
from jax.experimental import pallas as pl
from jax.experimental.pallas import tpu as pltpu
from jax.experimental.pallas import tpu_sc as plsc

VOCAB = 1_000_000
D = 64
BATCH = 4096
HIST = 200
NC, NS = 2, 16
NW = NC * NS
ROWS_PW = BATCH // NW
S0, S1 = 104, 96
NBUF = 4
NCHUNK = 2 * ROWS_PW
GROUPS = NCHUNK // NBUF

_SIZE = (S0, S1)
_OFF = (0, S0)

TBLK = 246


def _transpose_body(tt_hbm, pairs_hbm, bufI, bufO, isem, osem):
    wid = lax.axis_index("s") * NC + lax.axis_index("c")

    C0MAX = (VOCAB // 128 - 1) * 128

    def c0_of(t):
        return pl.multiple_of(jnp.minimum(128 * (wid + NW * t), C0MAX), 128)

    def in_desc(t, b):
        return pltpu.make_async_copy(
            tt_hbm.at[:, pl.ds(c0_of(t), 128)], bufI.at[b], isem.at[b]
        )

    def out_desc(t, b):
        return pltpu.make_async_copy(
            bufO.at[b],
            pairs_hbm.at[pl.ds(pl.multiple_of(c0_of(t) >> 1, 8), D), :],
            osem.at[b],
        )

    iota16 = lax.iota(jnp.int32, 16)

    def transpose(b):
        def prow(p, carry):
            for h in range(2):
                col = jnp.full((16,), 2 * p + h, jnp.int32)
                for c in range(D // 16):
                    val = plsc.load_gather(bufI.at[b], [iota16 + 16 * c, col])
                    bufO[b, p, pl.ds(16 * c + D * h, 16)] = val
            return carry

        lax.fori_loop(0, D, prow, 0)

    in_desc(0, 0).start()
    in_desc(1, 1).start()

    def pair(tt, carry):
        for h in range(2):
            t = 2 * tt + h
            b = h
            in_desc(t, b).wait()

            @pl.when(t >= 2)
            def _():
                out_desc(t - 2, b).wait()

            transpose(b)
            out_desc(t, b).start()

            @pl.when(t + 2 < TBLK)
            def _():
                in_desc(t + 2, b).start()

        return carry

    lax.fori_loop(0, TBLK // 2, pair, 0)
    out_desc(TBLK - 2, 0).wait()
    out_desc(TBLK - 1, 1).wait()

    @pl.when(wid == 0)
    def _():
        tc0 = VOCAB - 64
        for d in range(D):
            pltpu.make_async_copy(
                tt_hbm.at[d, pl.ds(tc0, 64)], bufI.at[0, d, pl.ds(0, 64)], isem.at[0]
            ).start()
        for d in range(D):
            pltpu.make_async_copy(
                tt_hbm.at[d, pl.ds(tc0, 64)], bufI.at[0, d, pl.ds(0, 64)], isem.at[0]
            ).wait()

        def prow(p, carry):
            for h in range(2):
                col = jnp.full((16,), 2 * p + h, jnp.int32)
                for c in range(D // 16):
                    val = plsc.load_gather(bufI.at[0], [iota16 + 16 * c, col])
                    bufO[0, p, pl.ds(16 * c + D * h, 16)] = val
            return carry

        lax.fori_loop(0, 32, prow, 0)
        tout = pltpu.make_async_copy(
            bufO.at[0, pl.ds(0, 32)],
            pairs_hbm.at[pl.ds(tc0 // 2, 32), :],
            osem.at[0],
        )
        tout.start()
        tout.wait()


def _body(x_hbm, table_hbm, out_hbm, idx_v, rows_v, gsem, osem):
    c = lax.axis_index("c")
    s = lax.axis_index("s")
    wid = s * NC + c
    b0 = wid * ROWS_PW
    pltpu.sync_copy(x_hbm.at[pl.ds(b0, ROWS_PW)], idx_v)

    def gather_desc(g, bank, b):
        j = g * (NBUF // 2) + (b >> 1)
        p = b & 1
        return pltpu.make_async_copy(
            table_hbm.at[idx_v.at[j, pl.ds(_OFF[p], _SIZE[p])]],
            rows_v.at[bank, b, pl.ds(0, _SIZE[p])],
            gsem.at[bank, b],
        )

    def write_desc(g, bank, b):
        j = g * (NBUF // 2) + (b >> 1)
        p = b & 1
        row0 = (b0 + j) * HIST + _OFF[p]
        return pltpu.make_async_copy(
            rows_v.at[bank, b, pl.ds(0, _SIZE[p])],
            out_hbm.at[pl.ds(row0, _SIZE[p]), pl.ds(0, D)],
            osem.at[bank, b],
        )

    for b in range(NBUF):
        gather_desc(0, 0, b).start()

    def pair(pp, carry):
        for h in range(2):
            g = 2 * pp + h
            bank = h
            for b in range(NBUF):

                @pl.when(g + 1 < GROUPS)
                def _():
                    @pl.when(g >= 1)
                    def _():
                        write_desc(g - 1, 1 - bank, b).wait()

                    gather_desc(g + 1, 1 - bank, b).start()

            for b in range(NBUF):
                gather_desc(g, bank, b).wait()
                write_desc(g, bank, b).start()
        return carry

    lax.fori_loop(0, GROUPS // 2, pair, 0)
    for b in range(NBUF):
        write_desc(GROUPS - 2, 0, b).wait()
        write_desc(GROUPS - 1, 1, b).wait()


@jax.jit
def kernel(x, table):
    mesh = plsc.VectorSubcoreMesh(
        core_axis_name="c", subcore_axis_name="s", num_cores=NC, num_subcores=NS
    )
    pairs = pl.kernel(
        _transpose_body,
        out_type=jax.ShapeDtypeStruct((VOCAB // 2, 2 * D), jnp.float32),
        mesh=mesh,
        scratch_types=[
            pltpu.VMEM((2, D, 128), jnp.float32),
            pltpu.VMEM((2, D, 128), jnp.float32),
            pltpu.SemaphoreType.DMA((2,)),
            pltpu.SemaphoreType.DMA((2,)),
        ],
        compiler_params=pltpu.CompilerParams(needs_layout_passes=False),
    )(table.T)
    table_lin = pairs.reshape(VOCAB, D)
    out128 = pl.kernel(
        _body,
        out_type=jax.ShapeDtypeStruct((BATCH * HIST, 2 * D), jnp.float32),
        mesh=mesh,
        scratch_types=[
            pltpu.VMEM((ROWS_PW, HIST), jnp.int32),
            pltpu.VMEM((2, NBUF, S0, D), jnp.float32),
            pltpu.SemaphoreType.DMA((2, NBUF)),
            pltpu.SemaphoreType.DMA((2, NBUF)),
        ],
        compiler_params=pltpu.CompilerParams(use_tc_tiling_on_sc=False),
    )(x, table_lin)
    return out128[:, :D].reshape(BATCH, HIST, D)


# speedup vs baseline: 1.9775x; 1.9775x over previous
import jax
import jax.numpy as jnp
from jax import lax
from jax.experimental import pallas as pl
from jax.experimental.pallas import tpu as pltpu
from jax.experimental.pallas import tpu_sc as plsc

VOCAB = 1_000_000
D = 64
BATCH = 4096
HIST = 200
NC, NS = 2, 16
NW = NC * NS
ROWS_PW = BATCH // NW
S0, S1 = 104, 96
NBUF = 4
NCHUNK = 2 * ROWS_PW
GROUPS = NCHUNK // NBUF

_SIZE = (S0, S1)
_OFF = (0, S0)


def _body(x_hbm, table_hbm, out_hbm, idx_v, rows_v, gsem, osem):
    c = lax.axis_index("c")
    s = lax.axis_index("s")
    wid = s * NC + c
    b0 = wid * ROWS_PW
    pltpu.sync_copy(x_hbm.at[pl.ds(b0, ROWS_PW)], idx_v)

    def gather_desc(g, bank, b):
        j = g * (NBUF // 2) + (b >> 1)
        p = b & 1
        return pltpu.make_async_copy(
            table_hbm.at[idx_v.at[j, pl.ds(_OFF[p], _SIZE[p])]],
            rows_v.at[bank, b, pl.ds(0, _SIZE[p])],
            gsem.at[bank, b],
        )

    def write_desc(g, bank, b):
        j = g * (NBUF // 2) + (b >> 1)
        p = b & 1
        row0 = (b0 + j) * HIST + _OFF[p]
        return pltpu.make_async_copy(
            rows_v.at[bank, b, pl.ds(0, _SIZE[p])],
            out_hbm.at[pl.ds(row0, _SIZE[p]), pl.ds(0, D)],
            osem.at[bank, b],
        )

    for b in range(NBUF):
        gather_desc(0, 0, b).start()

    def pair(pp, carry):
        for h in range(2):
            g = 2 * pp + h
            bank = h
            for b in range(NBUF):

                @pl.when(g + 1 < GROUPS)
                def _():
                    @pl.when(g >= 1)
                    def _():
                        write_desc(g - 1, 1 - bank, b).wait()

                    gather_desc(g + 1, 1 - bank, b).start()

            for b in range(NBUF):
                gather_desc(g, bank, b).wait()
                write_desc(g, bank, b).start()
        return carry

    lax.fori_loop(0, GROUPS // 2, pair, 0)
    for b in range(NBUF):
        write_desc(GROUPS - 2, 0, b).wait()
        write_desc(GROUPS - 1, 1, b).wait()


@jax.jit
def kernel(x, table):
    mesh = plsc.VectorSubcoreMesh(
        core_axis_name="c", subcore_axis_name="s", num_cores=NC, num_subcores=NS
    )
    out128 = pl.kernel(
        _body,
        out_type=jax.ShapeDtypeStruct((BATCH * HIST, 2 * D), jnp.float32),
        mesh=mesh,
        scratch_types=[
            pltpu.VMEM((ROWS_PW, HIST), jnp.int32),
            pltpu.VMEM((2, NBUF, S0, D), jnp.float32),
            pltpu.SemaphoreType.DMA((2, NBUF)),
            pltpu.SemaphoreType.DMA((2, NBUF)),
        ],
        compiler_params=pltpu.CompilerParams(use_tc_tiling_on_sc=False),
    )(x, table)
    return out128[:, :D].reshape(BATCH, HIST, D)
